# baseline (speedup 1.0000x reference)
import functools

import jax
import jax.numpy as jnp
from jax import lax
from jax.experimental import pallas as pl
from jax.experimental.pallas import tpu as pltpu
from jax.experimental.pallas import tpu_sc as plsc

_N = 10000
_E = 320000
_D_IN = 128
_D_HID = 128
_D_OUT = 32

_NC = 2
_NS = 16
_NW = _NC * _NS
_E_PER = _E // _NW
_CHUNK = 80
_ITERS = _E_PER // _CHUNK
_RPT = _N // _NS


_NBUF = 3
_G = (_ITERS - _NBUF) // _NBUF
_REM = _ITERS - _NBUF * _G


def _make_sc_aggregate(D):
    mesh = plsc.VectorSubcoreMesh(core_axis_name="c", subcore_axis_name="s")

    @functools.partial(
        pl.kernel,
        out_type=jax.ShapeDtypeStruct((_NC, _N, D), jnp.float32),
        mesh=mesh,
        scratch_types=(
            [pltpu.VMEM((_ITERS, _CHUNK), jnp.int32)] * 2
            + [pltpu.VMEM((_CHUNK, D), jnp.float32)] * _NBUF
            + [pltpu.VMEM_SHARED((_N, D), jnp.float32)]
            + [pltpu.SemaphoreType.DMA] * (2 * _NBUF)
        ),
        compiler_params=pltpu.CompilerParams(use_tc_tiling_on_sc=False),
    )
    def agg(h_hbm, ei_hbm, out_hbm, sidx_v, didx_v, *rest):
        rows = rest[:_NBUF]
        acc_s = rest[_NBUF]
        gsem = rest[_NBUF + 1:_NBUF + 1 + _NBUF]
        ssem = rest[_NBUF + 1 + _NBUF:]
        cid = lax.axis_index("c")
        sid = lax.axis_index("s")
        wid = sid * _NC + cid
        r0 = sid * _RPT
        pltpu.sync_copy(h_hbm.at[pl.ds(r0, _RPT)], acc_s.at[pl.ds(r0, _RPT)])
        pltpu.sync_copy(ei_hbm.at[0, wid], sidx_v)
        pltpu.sync_copy(ei_hbm.at[1, wid], didx_v)
        plsc.subcore_barrier()

        def gather(i, b):
            pltpu.async_copy(h_hbm.at[sidx_v.at[i]], rows[b], gsem[b])

        def wait_gather(i, b):
            pltpu.make_async_copy(h_hbm.at[sidx_v.at[i]],
                                  rows[b], gsem[b]).wait()

        def scatter(i, b):
            pltpu.async_copy(rows[b], acc_s.at[didx_v.at[i]],
                             ssem[b], add=True)

        def wait_scatter(i, b):
            pltpu.make_async_copy(rows[b], acc_s.at[didx_v.at[i]],
                                  ssem[b]).wait()

        for b in range(_NBUF):
            gather(b, b)

        def group(j, carry):
            i0 = j * _NBUF
            for b in range(_NBUF):
                wait_gather(i0 + b, b)
                scatter(i0 + b, b)
            for b in range(_NBUF):
                wait_scatter(i0 + b, b)
                gather(i0 + _NBUF + b, b)
            return carry

        lax.fori_loop(0, _G, group, 0)
        t0 = _NBUF * _G
        for b in range(_NBUF):
            wait_gather(t0 + b, b)
            scatter(t0 + b, b)
        for i in range(t0 + _NBUF, _ITERS):
            b = i % _NBUF
            wait_scatter(i - _NBUF, b)
            gather(i, b)
            wait_gather(i, b)
            scatter(i, b)
        for i in range(_ITERS - _NBUF, _ITERS):
            wait_scatter(i, i % _NBUF)
        plsc.subcore_barrier()
        pltpu.sync_copy(acc_s.at[pl.ds(r0, _RPT)],
                        out_hbm.at[cid, pl.ds(r0, _RPT)])

    return agg


_sc_agg = _make_sc_aggregate(_D_HID)


def _bn(z, g, b):
    mean = jnp.mean(z, axis=0, keepdims=True)
    zc = z - mean
    var = jnp.mean(zc * zc, axis=0, keepdims=True)
    return zc * lax.rsqrt(var + 1e-5) * g + b


def _tc_block1(x, parts, W1a, b1a, W1b, b1b, g1, be1):
    def body(x_ref, parts_ref, W1a_ref, b1a_ref, W1b_ref, b1b_ref,
             g1_ref, be1_ref, h1_ref):
        s = parts_ref[0] + parts_ref[1] - x_ref[...]
        z = jnp.dot(s, W1a_ref[...], preferred_element_type=jnp.float32)
        z = jnp.maximum(z + b1a_ref[...], 0.0)
        z = jnp.dot(z, W1b_ref[...], preferred_element_type=jnp.float32)
        z = z + b1b_ref[...]
        h1 = _bn(z, g1_ref[...], be1_ref[...])
        h1_ref[...] = jnp.maximum(h1, 0.0)

    return pl.pallas_call(
        body,
        out_shape=jax.ShapeDtypeStruct((_N, _D_HID), jnp.float32),
    )(x, parts, W1a, b1a, W1b, b1b, g1, be1)


def _tc_block2(h1, parts2, W2a, b2a, W2b, b2b, g2, be2):
    def body(h1_ref, parts2_ref, W2a_ref, b2a_ref, W2b_ref, b2b_ref,
             g2_ref, be2_ref, out_ref):
        s = parts2_ref[0] + parts2_ref[1] - h1_ref[...]
        z = jnp.dot(s, W2a_ref[...], preferred_element_type=jnp.float32)
        z = jnp.maximum(z + b2a_ref[...], 0.0)
        z = jnp.dot(z, W2b_ref[...], preferred_element_type=jnp.float32)
        z = z + b2b_ref[...]
        h2 = _bn(z, g2_ref[...], be2_ref[...])
        out_ref[...] = jnp.maximum(h2, 0.0)

    return pl.pallas_call(
        body,
        out_shape=jax.ShapeDtypeStruct((_N, _D_OUT), jnp.float32),
    )(h1, parts2, W2a, b2a, W2b, b2b, g2, be2)


def kernel(x, edge_index, W1a, b1a, W1b, b1b, g1, be1,
           W2a, b2a, W2b, b2b, g2, be2):
    ei = edge_index.reshape(2, _NW, _ITERS, _CHUNK)
    parts1 = _sc_agg(x, ei)
    h1 = _tc_block1(x, parts1,
                    W1a, b1a.reshape(1, -1), W1b, b1b.reshape(1, -1),
                    g1.reshape(1, -1), be1.reshape(1, -1))
    parts2 = _sc_agg(h1, ei)
    return _tc_block2(h1, parts2,
                      W2a, b2a.reshape(1, -1), W2b, b2b.reshape(1, -1),
                      g2.reshape(1, -1), be2.reshape(1, -1))


# speedup vs baseline: 1.0677x; 1.0677x over previous
import functools

import jax
import jax.numpy as jnp
from jax import lax
from jax.experimental import pallas as pl
from jax.experimental.pallas import tpu as pltpu
from jax.experimental.pallas import tpu_sc as plsc

_N = 10000
_E = 320000
_D_IN = 128
_D_HID = 128
_D_OUT = 32

_NC = 2
_NS = 16
_NW = _NC * _NS
_E_PER = _E // _NW
_CHUNK = 40
_ITERS = _E_PER // _CHUNK
_RPT = _N // _NS


_NBUF = 6
_G = (_ITERS - _NBUF) // _NBUF
_REM = _ITERS - _NBUF * _G


def _make_sc_aggregate(D):
    mesh = plsc.VectorSubcoreMesh(core_axis_name="c", subcore_axis_name="s")

    @functools.partial(
        pl.kernel,
        out_type=jax.ShapeDtypeStruct((_NC, _N, D), jnp.float32),
        mesh=mesh,
        scratch_types=(
            [pltpu.VMEM((_ITERS, _CHUNK), jnp.int32)] * 2
            + [pltpu.VMEM((_CHUNK, D), jnp.float32)] * _NBUF
            + [pltpu.VMEM_SHARED((_N, D), jnp.float32)]
            + [pltpu.SemaphoreType.DMA] * (2 * _NBUF)
        ),
        compiler_params=pltpu.CompilerParams(use_tc_tiling_on_sc=False),
    )
    def agg(h_hbm, ei_hbm, out_hbm, sidx_v, didx_v, *rest):
        rows = rest[:_NBUF]
        acc_s = rest[_NBUF]
        gsem = rest[_NBUF + 1:_NBUF + 1 + _NBUF]
        ssem = rest[_NBUF + 1 + _NBUF:]
        cid = lax.axis_index("c")
        sid = lax.axis_index("s")
        wid = sid * _NC + cid
        r0 = sid * _RPT
        pltpu.sync_copy(h_hbm.at[pl.ds(r0, _RPT)], acc_s.at[pl.ds(r0, _RPT)])
        pltpu.sync_copy(ei_hbm.at[0, wid], sidx_v)
        pltpu.sync_copy(ei_hbm.at[1, wid], didx_v)
        plsc.subcore_barrier()

        def gather(i, b):
            pltpu.async_copy(h_hbm.at[sidx_v.at[i]], rows[b], gsem[b])

        def wait_gather(i, b):
            pltpu.make_async_copy(h_hbm.at[sidx_v.at[i]],
                                  rows[b], gsem[b]).wait()

        def scatter(i, b):
            pltpu.async_copy(rows[b], acc_s.at[didx_v.at[i]],
                             ssem[b], add=True)

        def wait_scatter(i, b):
            pltpu.make_async_copy(rows[b], acc_s.at[didx_v.at[i]],
                                  ssem[b]).wait()

        for b in range(_NBUF):
            gather(b, b)

        def group(j, carry):
            i0 = j * _NBUF
            for b in range(_NBUF):
                wait_gather(i0 + b, b)
                scatter(i0 + b, b)
            for b in range(_NBUF):
                wait_scatter(i0 + b, b)
                gather(i0 + _NBUF + b, b)
            return carry

        lax.fori_loop(0, _G, group, 0)
        t0 = _NBUF * _G
        for b in range(_NBUF):
            wait_gather(t0 + b, b)
            scatter(t0 + b, b)
        for i in range(t0 + _NBUF, _ITERS):
            b = i % _NBUF
            wait_scatter(i - _NBUF, b)
            gather(i, b)
            wait_gather(i, b)
            scatter(i, b)
        for i in range(_ITERS - _NBUF, _ITERS):
            wait_scatter(i, i % _NBUF)
        plsc.subcore_barrier()
        pltpu.sync_copy(acc_s.at[pl.ds(r0, _RPT)],
                        out_hbm.at[cid, pl.ds(r0, _RPT)])

    return agg


_sc_agg = _make_sc_aggregate(_D_HID)


def _bn(z, g, b):
    mean = jnp.mean(z, axis=0, keepdims=True)
    zc = z - mean
    var = jnp.mean(zc * zc, axis=0, keepdims=True)
    return zc * lax.rsqrt(var + 1e-5) * g + b


def _tc_block1(x, parts, W1a, b1a, W1b, b1b, g1, be1):
    def body(x_ref, parts_ref, W1a_ref, b1a_ref, W1b_ref, b1b_ref,
             g1_ref, be1_ref, h1_ref):
        s = parts_ref[0] + parts_ref[1] - x_ref[...]
        z = jnp.dot(s, W1a_ref[...], preferred_element_type=jnp.float32)
        z = jnp.maximum(z + b1a_ref[...], 0.0)
        z = jnp.dot(z, W1b_ref[...], preferred_element_type=jnp.float32)
        z = z + b1b_ref[...]
        h1 = _bn(z, g1_ref[...], be1_ref[...])
        h1_ref[...] = jnp.maximum(h1, 0.0)

    return pl.pallas_call(
        body,
        out_shape=jax.ShapeDtypeStruct((_N, _D_HID), jnp.float32),
    )(x, parts, W1a, b1a, W1b, b1b, g1, be1)


def _tc_block2(h1, parts2, W2a, b2a, W2b, b2b, g2, be2):
    def body(h1_ref, parts2_ref, W2a_ref, b2a_ref, W2b_ref, b2b_ref,
             g2_ref, be2_ref, out_ref):
        s = parts2_ref[0] + parts2_ref[1] - h1_ref[...]
        z = jnp.dot(s, W2a_ref[...], preferred_element_type=jnp.float32)
        z = jnp.maximum(z + b2a_ref[...], 0.0)
        z = jnp.dot(z, W2b_ref[...], preferred_element_type=jnp.float32)
        z = z + b2b_ref[...]
        h2 = _bn(z, g2_ref[...], be2_ref[...])
        out_ref[...] = jnp.maximum(h2, 0.0)

    return pl.pallas_call(
        body,
        out_shape=jax.ShapeDtypeStruct((_N, _D_OUT), jnp.float32),
    )(h1, parts2, W2a, b2a, W2b, b2b, g2, be2)


def kernel(x, edge_index, W1a, b1a, W1b, b1b, g1, be1,
           W2a, b2a, W2b, b2b, g2, be2):
    ei = edge_index.reshape(2, _NW, _ITERS, _CHUNK)
    parts1 = _sc_agg(x, ei)
    h1 = _tc_block1(x, parts1,
                    W1a, b1a.reshape(1, -1), W1b, b1b.reshape(1, -1),
                    g1.reshape(1, -1), be1.reshape(1, -1))
    parts2 = _sc_agg(h1, ei)
    return _tc_block2(h1, parts2,
                      W2a, b2a.reshape(1, -1), W2b, b2b.reshape(1, -1),
                      g2.reshape(1, -1), be2.reshape(1, -1))
